# Initial kernel scaffold; baseline (speedup 1.0000x reference)
import jax
import jax.numpy as jnp
from jax.experimental import pallas as pl


def kernel(x, edge_index, W1, b1, W2, b2):
    raise NotImplementedError("write your pallas kernel here")


# speedup vs baseline: 37.8523x; 37.8523x over previous
import functools

import jax
import jax.numpy as jnp
from jax import lax
from jax.experimental import pallas as pl
from jax.experimental.pallas import tpu as pltpu, tpu_sc as plsc

N = 10000
NPAD = 10240
E = 320000
D_FEAT = 128
D_HID = 16
N_CLASSES = 40

CHUNK = 128
NWORK = 32
CW = 80
NCHUNK = NWORK * CW
EPAD = NCHUNK * CHUNK
SLAB = NPAD // 16

_mesh = plsc.VectorSubcoreMesh(core_axis_name="c", subcore_axis_name="s")
_sc_params = pltpu.CompilerParams(use_tc_tiling_on_sc=False)


def _make_deg():
  @functools.partial(
      pl.kernel,
      mesh=_mesh,
      out_type=jax.ShapeDtypeStruct((2, NPAD, 1), jnp.float32),
      compiler_params=_sc_params,
      scratch_types=[
          pltpu.VMEM((CW, CHUNK), jnp.int32),
          pltpu.VMEM((CHUNK, 1), jnp.float32),
          pltpu.VMEM_SHARED((NPAD, 1), jnp.float32),
      ],
  )
  def deg_kernel(dst_hbm, ones_hbm, z_hbm, out_hbm, didx, ones_v, acc):
    c = lax.axis_index("c")
    s = lax.axis_index("s")
    wid = s * 2 + c
    pltpu.sync_copy(z_hbm.at[pl.ds(s * SLAB, SLAB)], acc.at[pl.ds(s * SLAB, SLAB)])
    pltpu.sync_copy(dst_hbm.at[pl.ds(wid * CW, CW)], didx)
    pltpu.sync_copy(ones_hbm, ones_v)
    plsc.subcore_barrier()

    @pl.loop(0, CW)
    def _(j):
      pltpu.sync_copy(ones_v, acc.at[didx.at[j]], add=True)

    plsc.subcore_barrier()
    pltpu.sync_copy(acc.at[pl.ds(s * SLAB, SLAB)],
                    out_hbm.at[c, pl.ds(s * SLAB, SLAB)])

  return deg_kernel


def _make_agg(d):
  @functools.partial(
      pl.kernel,
      mesh=_mesh,
      out_type=jax.ShapeDtypeStruct((2, NPAD, d), jnp.float32),
      compiler_params=_sc_params,
      scratch_types=[
          pltpu.VMEM((CW, CHUNK), jnp.int32),
          pltpu.VMEM((CW, CHUNK), jnp.int32),
          pltpu.VMEM((CHUNK, d), jnp.float32),
          pltpu.VMEM((CHUNK, d), jnp.float32),
          pltpu.VMEM_SHARED((NPAD, d), jnp.float32),
          pltpu.SemaphoreType.DMA,
          pltpu.SemaphoreType.DMA,
      ],
  )
  def agg_kernel(src_hbm, dst_hbm, g_hbm, z_hbm, out_hbm,
                 sidx, didx, buf0, buf1, acc, sem0, sem1):
    c = lax.axis_index("c")
    s = lax.axis_index("s")
    wid = s * 2 + c
    pltpu.sync_copy(z_hbm.at[pl.ds(s * SLAB, SLAB)], acc.at[pl.ds(s * SLAB, SLAB)])
    pltpu.sync_copy(src_hbm.at[pl.ds(wid * CW, CW)], sidx)
    pltpu.sync_copy(dst_hbm.at[pl.ds(wid * CW, CW)], didx)
    plsc.subcore_barrier()

    pltpu.async_copy(g_hbm.at[sidx.at[0]], buf0, sem0)
    pltpu.async_copy(g_hbm.at[sidx.at[1]], buf1, sem1)

    @pl.loop(0, CW // 2 - 1)
    def _(jj):
      j0 = jj * 2
      pltpu.make_async_copy(g_hbm.at[sidx.at[j0]], buf0, sem0).wait()
      pltpu.sync_copy(buf0, acc.at[didx.at[j0]], add=True)
      pltpu.async_copy(g_hbm.at[sidx.at[j0 + 2]], buf0, sem0)
      pltpu.make_async_copy(g_hbm.at[sidx.at[j0 + 1]], buf1, sem1).wait()
      pltpu.sync_copy(buf1, acc.at[didx.at[j0 + 1]], add=True)
      pltpu.async_copy(g_hbm.at[sidx.at[j0 + 3]], buf1, sem1)

    pltpu.make_async_copy(g_hbm.at[sidx.at[CW - 2]], buf0, sem0).wait()
    pltpu.sync_copy(buf0, acc.at[didx.at[CW - 2]], add=True)
    pltpu.make_async_copy(g_hbm.at[sidx.at[CW - 1]], buf1, sem1).wait()
    pltpu.sync_copy(buf1, acc.at[didx.at[CW - 1]], add=True)

    plsc.subcore_barrier()
    pltpu.sync_copy(acc.at[pl.ds(s * SLAB, SLAB)],
                    out_hbm.at[c, pl.ds(s * SLAB, SLAB)])

  return agg_kernel


_deg = _make_deg()
_agg16 = _make_agg(D_HID)
_agg40 = _make_agg(N_CLASSES)


def _mm1_body(x_ref, w_ref, o_ref):
  o_ref[...] = jnp.dot(x_ref[...], w_ref[...],
                       preferred_element_type=jnp.float32)


def _tc_mm1(x_pad, w1):
  return pl.pallas_call(
      _mm1_body,
      grid=(8,),
      in_specs=[
          pl.BlockSpec((NPAD // 8, D_FEAT), lambda i: (i, 0)),
          pl.BlockSpec((D_FEAT, D_HID), lambda i: (0, 0)),
      ],
      out_specs=pl.BlockSpec((NPAD // 8, D_HID), lambda i: (i, 0)),
      out_shape=jax.ShapeDtypeStruct((NPAD, D_HID), jnp.float32),
  )(x_pad, w1)


def _scale1_body(d0_ref, d1_ref, h_ref, dis_ref, g1_ref):
  deg = d0_ref[...] + d1_ref[...] + 1.0
  dis = lax.rsqrt(deg)
  dis_ref[...] = dis
  g1_ref[...] = dis * h_ref[...]


def _tc_scale1(d0, d1, h):
  return pl.pallas_call(
      _scale1_body,
      out_shape=(
          jax.ShapeDtypeStruct((NPAD, 1), jnp.float32),
          jax.ShapeDtypeStruct((NPAD, D_HID), jnp.float32),
      ),
  )(d0, d1, h)


def _layer2_body(a0_ref, a1_ref, g1_ref, dis_ref, b1_ref, w2_ref, g2_ref):
  s = dis_ref[...] * (a0_ref[...] + a1_ref[...] + g1_ref[...]) + b1_ref[...]
  h1 = jnp.maximum(s, 0.0)
  g2_ref[...] = dis_ref[...] * jnp.dot(h1, w2_ref[...],
                                       preferred_element_type=jnp.float32)


def _tc_layer2(a0, a1, g1, dis, b1, w2):
  return pl.pallas_call(
      _layer2_body,
      out_shape=jax.ShapeDtypeStruct((NPAD, N_CLASSES), jnp.float32),
  )(a0, a1, g1, dis, b1, w2)


def _final_body(q0_ref, q1_ref, g2_ref, dis_ref, b2_ref, o_ref):
  l = dis_ref[...] * (q0_ref[...] + q1_ref[...] + g2_ref[...]) + b2_ref[...]
  m = jnp.max(l, axis=1, keepdims=True)
  e = jnp.exp(l - m)
  lse = jnp.log(jnp.sum(e, axis=1, keepdims=True)) + m
  o_ref[...] = l - lse


def _tc_final(q0, q1, g2, dis, b2):
  return pl.pallas_call(
      _final_body,
      out_shape=jax.ShapeDtypeStruct((NPAD, N_CLASSES), jnp.float32),
  )(q0, q1, g2, dis, b2)


def kernel(x, edge_index, W1, b1, W2, b2):
  src = edge_index[0].astype(jnp.int32)
  dst = edge_index[1].astype(jnp.int32)
  pad = N + (jnp.arange(EPAD - E, dtype=jnp.int32) % (NPAD - N))
  srcp = jnp.concatenate([src, pad]).reshape(NCHUNK, CHUNK)
  dstp = jnp.concatenate([dst, pad]).reshape(NCHUNK, CHUNK)
  x_pad = jnp.pad(x, ((0, NPAD - N), (0, 0)))

  z1 = jnp.zeros((NPAD, 1), jnp.float32)
  z16 = jnp.zeros((NPAD, D_HID), jnp.float32)
  z40 = jnp.zeros((NPAD, N_CLASSES), jnp.float32)
  ones = jnp.ones((CHUNK, 1), jnp.float32)

  degp = _deg(dstp, ones, z1)
  h = _tc_mm1(x_pad, W1)
  dis, g1 = _tc_scale1(degp[0], degp[1], h)
  a = _agg16(srcp, dstp, g1, z16)
  g2 = _tc_layer2(a[0], a[1], g1, dis,
                  b1.reshape(1, D_HID), W2)
  q = _agg40(srcp, dstp, g2, z40)
  out = _tc_final(q[0], q[1], g2, dis,
                  b2.reshape(1, N_CLASSES))
  return out[:N]


# speedup vs baseline: 43.9636x; 1.1615x over previous
import functools

import jax
import jax.numpy as jnp
from jax import lax
from jax.experimental import pallas as pl
from jax.experimental.pallas import tpu as pltpu, tpu_sc as plsc

N = 10000
E = 320000
D_FEAT = 128
D_HID = 16
N_CLASSES = 40

CHUNK = 128
NWORK = 32
NCHUNK = E // CHUNK
CW_LO = NCHUNK // NWORK
N_HI = NCHUNK - CW_LO * NWORK
CW_HI = CW_LO + 1
SLAB = N // 16

_mesh = plsc.VectorSubcoreMesh(core_axis_name="c", subcore_axis_name="s")
_sc_params = pltpu.CompilerParams(use_tc_tiling_on_sc=False)


def _worker_chunks(c, s):
  wid = s * 2 + c
  base = CW_LO * wid + jnp.minimum(wid, N_HI)
  has_extra = wid < N_HI
  return wid, base, has_extra


def _make_deg():
  @functools.partial(
      pl.kernel,
      mesh=_mesh,
      out_type=jax.ShapeDtypeStruct((2, N, 1), jnp.float32),
      compiler_params=_sc_params,
      scratch_types=[
          pltpu.VMEM((CW_HI * CHUNK,), jnp.int32),
          pltpu.VMEM((CHUNK, 1), jnp.float32),
          pltpu.VMEM_SHARED((N, 1), jnp.float32),
      ],
  )
  def deg_kernel(ei_hbm, ones_hbm, z_hbm, out_hbm, didx, ones_v, acc):
    c = lax.axis_index("c")
    s = lax.axis_index("s")
    wid, base, has_extra = _worker_chunks(c, s)
    pltpu.sync_copy(z_hbm.at[pl.ds(s * SLAB, SLAB)], acc.at[pl.ds(s * SLAB, SLAB)])
    pltpu.sync_copy(ei_hbm.at[1, pl.ds(base * CHUNK, CW_LO * CHUNK)],
                    didx.at[pl.ds(0, CW_LO * CHUNK)])

    @pl.when(has_extra)
    def _():
      pltpu.sync_copy(
          ei_hbm.at[1, pl.ds((base + CW_LO) * CHUNK, CHUNK)],
          didx.at[pl.ds(CW_LO * CHUNK, CHUNK)])

    pltpu.sync_copy(ones_hbm, ones_v)
    plsc.subcore_barrier()

    @pl.loop(0, CW_LO)
    def _(j):
      pltpu.sync_copy(ones_v, acc.at[didx.at[pl.ds(j * CHUNK, CHUNK)]], add=True)

    @pl.when(has_extra)
    def _():
      pltpu.sync_copy(ones_v, acc.at[didx.at[pl.ds(CW_LO * CHUNK, CHUNK)]],
                      add=True)

    plsc.subcore_barrier()
    pltpu.sync_copy(acc.at[pl.ds(s * SLAB, SLAB)],
                    out_hbm.at[c, pl.ds(s * SLAB, SLAB)])

  return deg_kernel


def _make_agg(d):
  @functools.partial(
      pl.kernel,
      mesh=_mesh,
      out_type=jax.ShapeDtypeStruct((2, N, d), jnp.float32),
      compiler_params=_sc_params,
      scratch_types=[
          pltpu.VMEM((CW_HI * CHUNK,), jnp.int32),
          pltpu.VMEM((CW_HI * CHUNK,), jnp.int32),
          pltpu.VMEM((CHUNK, d), jnp.float32),
          pltpu.VMEM((CHUNK, d), jnp.float32),
          pltpu.VMEM_SHARED((N, d), jnp.float32),
          pltpu.SemaphoreType.DMA,
          pltpu.SemaphoreType.DMA,
      ],
  )
  def agg_kernel(ei_hbm, g_hbm, z_hbm, out_hbm,
                 sidx, didx, buf0, buf1, acc, sem0, sem1):
    c = lax.axis_index("c")
    s = lax.axis_index("s")
    wid, base, has_extra = _worker_chunks(c, s)
    pltpu.sync_copy(z_hbm.at[pl.ds(s * SLAB, SLAB)], acc.at[pl.ds(s * SLAB, SLAB)])
    pltpu.sync_copy(ei_hbm.at[0, pl.ds(base * CHUNK, CW_LO * CHUNK)],
                    sidx.at[pl.ds(0, CW_LO * CHUNK)])
    pltpu.sync_copy(ei_hbm.at[1, pl.ds(base * CHUNK, CW_LO * CHUNK)],
                    didx.at[pl.ds(0, CW_LO * CHUNK)])

    @pl.when(has_extra)
    def _():
      pltpu.sync_copy(ei_hbm.at[0, pl.ds((base + CW_LO) * CHUNK, CHUNK)],
                      sidx.at[pl.ds(CW_LO * CHUNK, CHUNK)])
      pltpu.sync_copy(ei_hbm.at[1, pl.ds((base + CW_LO) * CHUNK, CHUNK)],
                      didx.at[pl.ds(CW_LO * CHUNK, CHUNK)])

    plsc.subcore_barrier()

    def gather(j, buf, sem):
      return pltpu.async_copy(g_hbm.at[sidx.at[pl.ds(j * CHUNK, CHUNK)]], buf, sem)

    def wait(j, buf, sem):
      pltpu.make_async_copy(g_hbm.at[sidx.at[pl.ds(j * CHUNK, CHUNK)]], buf, sem).wait()

    def scat(j, buf):
      pltpu.sync_copy(buf, acc.at[didx.at[pl.ds(j * CHUNK, CHUNK)]], add=True)

    gather(0, buf0, sem0)
    gather(1, buf1, sem1)

    @pl.loop(0, (CW_LO - 2) // 2)
    def _(jj):
      j0 = jj * 2
      wait(j0, buf0, sem0)
      scat(j0, buf0)
      gather(j0 + 2, buf0, sem0)
      wait(j0 + 1, buf1, sem1)
      scat(j0 + 1, buf1)
      gather(j0 + 3, buf1, sem1)

    wait(CW_LO - 2, buf0, sem0)
    scat(CW_LO - 2, buf0)

    @pl.when(has_extra)
    def _():
      gather(CW_LO, buf0, sem0)

    wait(CW_LO - 1, buf1, sem1)
    scat(CW_LO - 1, buf1)

    @pl.when(has_extra)
    def _():
      wait(CW_LO, buf0, sem0)
      scat(CW_LO, buf0)

    plsc.subcore_barrier()
    pltpu.sync_copy(acc.at[pl.ds(s * SLAB, SLAB)],
                    out_hbm.at[c, pl.ds(s * SLAB, SLAB)])

  return agg_kernel


_deg = _make_deg()
_agg16 = _make_agg(D_HID)
_agg40 = _make_agg(N_CLASSES)


def _mm1_body(x_ref, w_ref, o_ref):
  o_ref[...] = jnp.dot(x_ref[...], w_ref[...],
                       preferred_element_type=jnp.float32)


def _tc_mm1(x, w1):
  return pl.pallas_call(
      _mm1_body,
      grid=(10,),
      in_specs=[
          pl.BlockSpec((N // 10, D_FEAT), lambda i: (i, 0)),
          pl.BlockSpec((D_FEAT, D_HID), lambda i: (0, 0)),
      ],
      out_specs=pl.BlockSpec((N // 10, D_HID), lambda i: (i, 0)),
      out_shape=jax.ShapeDtypeStruct((N, D_HID), jnp.float32),
  )(x, w1)


def _scale1_body(dp_ref, h_ref, dis_ref, g1_ref):
  deg = dp_ref[0] + dp_ref[1] + 1.0
  dis = lax.rsqrt(deg)
  dis_ref[...] = dis
  g1_ref[...] = dis * h_ref[...]


def _tc_scale1(dp, h):
  return pl.pallas_call(
      _scale1_body,
      out_shape=(
          jax.ShapeDtypeStruct((N, 1), jnp.float32),
          jax.ShapeDtypeStruct((N, D_HID), jnp.float32),
      ),
  )(dp, h)


def _layer2_body(a_ref, g1_ref, dis_ref, b1_ref, w2_ref, g2_ref):
  s = dis_ref[...] * (a_ref[0] + a_ref[1] + g1_ref[...]) + b1_ref[...]
  h1 = jnp.maximum(s, 0.0)
  g2_ref[...] = dis_ref[...] * jnp.dot(h1, w2_ref[...],
                                       preferred_element_type=jnp.float32)


def _tc_layer2(a, g1, dis, b1, w2):
  return pl.pallas_call(
      _layer2_body,
      out_shape=jax.ShapeDtypeStruct((N, N_CLASSES), jnp.float32),
  )(a, g1, dis, b1, w2)


def _final_body(q_ref, g2_ref, dis_ref, b2_ref, o_ref):
  l = dis_ref[...] * (q_ref[0] + q_ref[1] + g2_ref[...]) + b2_ref[...]
  m = jnp.max(l, axis=1, keepdims=True)
  e = jnp.exp(l - m)
  lse = jnp.log(jnp.sum(e, axis=1, keepdims=True)) + m
  o_ref[...] = l - lse


def _tc_final(q, g2, dis, b2):
  return pl.pallas_call(
      _final_body,
      out_shape=jax.ShapeDtypeStruct((N, N_CLASSES), jnp.float32),
  )(q, g2, dis, b2)


def kernel(x, edge_index, W1, b1, W2, b2):
  ei = edge_index.astype(jnp.int32)

  z1 = jnp.zeros((N, 1), jnp.float32)
  z16 = jnp.zeros((N, D_HID), jnp.float32)
  z40 = jnp.zeros((N, N_CLASSES), jnp.float32)
  ones = jnp.ones((CHUNK, 1), jnp.float32)

  degp = _deg(ei, ones, z1)
  h = _tc_mm1(x, W1)
  dis, g1 = _tc_scale1(degp, h)
  a = _agg16(ei, g1, z16)
  g2 = _tc_layer2(a, g1, dis,
                  b1.reshape(1, D_HID), W2)
  q = _agg40(ei, g2, z40)
  return _tc_final(q, g2, dis, b2.reshape(1, N_CLASSES))


# speedup vs baseline: 48.2897x; 1.0984x over previous
import functools

import jax
import jax.numpy as jnp
from jax import lax
from jax.experimental import pallas as pl
from jax.experimental.pallas import tpu as pltpu, tpu_sc as plsc

N = 10000
E = 320000
D_FEAT = 128
D_HID = 16
N_CLASSES = 40

CHUNK = 128
NWORK = 32
NCHUNK = E // CHUNK
CW_LO = NCHUNK // NWORK
N_HI = NCHUNK - CW_LO * NWORK
CW_HI = CW_LO + 1
SLAB = N // 16

_mesh = plsc.VectorSubcoreMesh(core_axis_name="c", subcore_axis_name="s")
_sc_params = pltpu.CompilerParams(use_tc_tiling_on_sc=False)


def _worker_chunks(c, s):
  wid = s * 2 + c
  base = CW_LO * wid + jnp.minimum(wid, N_HI)
  has_extra = wid < N_HI
  return wid, base, has_extra


def _make_deg():
  @functools.partial(
      pl.kernel,
      mesh=_mesh,
      out_type=jax.ShapeDtypeStruct((2, N, 1), jnp.float32),
      compiler_params=_sc_params,
      scratch_types=[
          pltpu.VMEM((CW_HI * CHUNK,), jnp.int32),
          pltpu.VMEM((CHUNK, 1), jnp.float32),
          pltpu.VMEM_SHARED((N, 1), jnp.float32),
      ],
  )
  def deg_kernel(ei_hbm, ones_hbm, z_hbm, out_hbm, didx, ones_v, acc):
    c = lax.axis_index("c")
    s = lax.axis_index("s")
    wid, base, has_extra = _worker_chunks(c, s)
    pltpu.sync_copy(z_hbm.at[pl.ds(s * SLAB, SLAB)], acc.at[pl.ds(s * SLAB, SLAB)])
    pltpu.sync_copy(ei_hbm.at[1, pl.ds(base * CHUNK, CW_LO * CHUNK)],
                    didx.at[pl.ds(0, CW_LO * CHUNK)])

    @pl.when(has_extra)
    def _():
      pltpu.sync_copy(
          ei_hbm.at[1, pl.ds((base + CW_LO) * CHUNK, CHUNK)],
          didx.at[pl.ds(CW_LO * CHUNK, CHUNK)])

    pltpu.sync_copy(ones_hbm, ones_v)
    plsc.subcore_barrier()

    @pl.loop(0, CW_LO)
    def _(j):
      pltpu.sync_copy(ones_v, acc.at[didx.at[pl.ds(j * CHUNK, CHUNK)]], add=True)

    @pl.when(has_extra)
    def _():
      pltpu.sync_copy(ones_v, acc.at[didx.at[pl.ds(CW_LO * CHUNK, CHUNK)]],
                      add=True)

    plsc.subcore_barrier()
    pltpu.sync_copy(acc.at[pl.ds(s * SLAB, SLAB)],
                    out_hbm.at[c, pl.ds(s * SLAB, SLAB)])

  return deg_kernel


def _make_agg(d):
  @functools.partial(
      pl.kernel,
      mesh=_mesh,
      out_type=jax.ShapeDtypeStruct((2, N, d), jnp.float32),
      compiler_params=_sc_params,
      scratch_types=[
          pltpu.VMEM((CW_HI * CHUNK,), jnp.int32),
          pltpu.VMEM((CW_HI * CHUNK,), jnp.int32),
          pltpu.VMEM((CHUNK, d), jnp.float32),
          pltpu.VMEM((CHUNK, d), jnp.float32),
          pltpu.VMEM_SHARED((N, d), jnp.float32),
          pltpu.SemaphoreType.DMA,
          pltpu.SemaphoreType.DMA,
      ],
  )
  def agg_kernel(ei_hbm, g_hbm, z_hbm, out_hbm,
                 sidx, didx, buf0, buf1, acc, sem0, sem1):
    c = lax.axis_index("c")
    s = lax.axis_index("s")
    wid, base, has_extra = _worker_chunks(c, s)
    pltpu.sync_copy(z_hbm.at[pl.ds(s * SLAB, SLAB)], acc.at[pl.ds(s * SLAB, SLAB)])
    pltpu.sync_copy(ei_hbm.at[0, pl.ds(base * CHUNK, CW_LO * CHUNK)],
                    sidx.at[pl.ds(0, CW_LO * CHUNK)])
    pltpu.sync_copy(ei_hbm.at[1, pl.ds(base * CHUNK, CW_LO * CHUNK)],
                    didx.at[pl.ds(0, CW_LO * CHUNK)])

    @pl.when(has_extra)
    def _():
      pltpu.sync_copy(ei_hbm.at[0, pl.ds((base + CW_LO) * CHUNK, CHUNK)],
                      sidx.at[pl.ds(CW_LO * CHUNK, CHUNK)])
      pltpu.sync_copy(ei_hbm.at[1, pl.ds((base + CW_LO) * CHUNK, CHUNK)],
                      didx.at[pl.ds(CW_LO * CHUNK, CHUNK)])

    plsc.subcore_barrier()

    def gather(j, buf, sem):
      return pltpu.async_copy(g_hbm.at[sidx.at[pl.ds(j * CHUNK, CHUNK)]], buf, sem)

    def wait(j, buf, sem):
      pltpu.make_async_copy(g_hbm.at[sidx.at[pl.ds(j * CHUNK, CHUNK)]], buf, sem).wait()

    def scat(j, buf):
      pltpu.sync_copy(buf, acc.at[didx.at[pl.ds(j * CHUNK, CHUNK)]], add=True)

    gather(0, buf0, sem0)
    gather(1, buf1, sem1)

    @pl.loop(0, (CW_LO - 2) // 2)
    def _(jj):
      j0 = jj * 2
      wait(j0, buf0, sem0)
      scat(j0, buf0)
      gather(j0 + 2, buf0, sem0)
      wait(j0 + 1, buf1, sem1)
      scat(j0 + 1, buf1)
      gather(j0 + 3, buf1, sem1)

    wait(CW_LO - 2, buf0, sem0)
    scat(CW_LO - 2, buf0)

    @pl.when(has_extra)
    def _():
      gather(CW_LO, buf0, sem0)

    wait(CW_LO - 1, buf1, sem1)
    scat(CW_LO - 1, buf1)

    @pl.when(has_extra)
    def _():
      wait(CW_LO, buf0, sem0)
      scat(CW_LO, buf0)

    plsc.subcore_barrier()
    pltpu.sync_copy(acc.at[pl.ds(s * SLAB, SLAB)],
                    out_hbm.at[c, pl.ds(s * SLAB, SLAB)])

  return agg_kernel


_deg = _make_deg()
_agg16 = _make_agg(D_HID)
_agg16b = _make_agg(D_HID)


def _mm1_body(x_ref, w_ref, o_ref):
  o_ref[...] = jnp.dot(x_ref[...], w_ref[...],
                       preferred_element_type=jnp.float32)


def _tc_mm1(x, w1):
  return pl.pallas_call(
      _mm1_body,
      grid=(10,),
      in_specs=[
          pl.BlockSpec((N // 10, D_FEAT), lambda i: (i, 0)),
          pl.BlockSpec((D_FEAT, D_HID), lambda i: (0, 0)),
      ],
      out_specs=pl.BlockSpec((N // 10, D_HID), lambda i: (i, 0)),
      out_shape=jax.ShapeDtypeStruct((N, D_HID), jnp.float32),
  )(x, w1)


def _scale1_body(dp_ref, h_ref, dis_ref, g1_ref):
  deg = dp_ref[0] + dp_ref[1] + 1.0
  dis = lax.rsqrt(deg)
  dis_ref[...] = dis
  g1_ref[...] = dis * h_ref[...]


def _tc_scale1(dp, h):
  return pl.pallas_call(
      _scale1_body,
      out_shape=(
          jax.ShapeDtypeStruct((N, 1), jnp.float32),
          jax.ShapeDtypeStruct((N, D_HID), jnp.float32),
      ),
  )(dp, h)


def _mid_body(a_ref, g1_ref, dis_ref, b1_ref, u_ref):
  s = dis_ref[...] * (a_ref[0] + a_ref[1] + g1_ref[...]) + b1_ref[...]
  u_ref[...] = dis_ref[...] * jnp.maximum(s, 0.0)


def _tc_mid(a, g1, dis, b1):
  return pl.pallas_call(
      _mid_body,
      out_shape=jax.ShapeDtypeStruct((N, D_HID), jnp.float32),
  )(a, g1, dis, b1)


def _final_body(q_ref, u_ref, dis_ref, w2_ref, b2_ref, o_ref):
  t = jnp.dot(q_ref[0] + q_ref[1] + u_ref[...], w2_ref[...],
              preferred_element_type=jnp.float32)
  l = dis_ref[...] * t + b2_ref[...]
  m = jnp.max(l, axis=1, keepdims=True)
  e = jnp.exp(l - m)
  lse = jnp.log(jnp.sum(e, axis=1, keepdims=True)) + m
  o_ref[...] = l - lse


def _tc_final(q, u, dis, w2, b2):
  return pl.pallas_call(
      _final_body,
      out_shape=jax.ShapeDtypeStruct((N, N_CLASSES), jnp.float32),
  )(q, u, dis, w2, b2)


def kernel(x, edge_index, W1, b1, W2, b2):
  ei = edge_index.astype(jnp.int32)

  z1 = jnp.zeros((N, 1), jnp.float32)
  z16 = jnp.zeros((N, D_HID), jnp.float32)
  ones = jnp.ones((CHUNK, 1), jnp.float32)

  degp = _deg(ei, ones, z1)
  h = _tc_mm1(x, W1)
  dis, g1 = _tc_scale1(degp, h)
  a = _agg16(ei, g1, z16)
  u = _tc_mid(a, g1, dis, b1.reshape(1, D_HID))
  q = _agg16b(ei, u, z16)
  return _tc_final(q, u, dis, W2, b2.reshape(1, N_CLASSES))


# speedup vs baseline: 48.9439x; 1.0135x over previous
import functools

import jax
import jax.numpy as jnp
from jax import lax
from jax.experimental import pallas as pl
from jax.experimental.pallas import tpu as pltpu, tpu_sc as plsc

N = 10000
E = 320000
D_FEAT = 128
D_HID = 16
N_CLASSES = 40

CHUNK = 128
NWORK = 32
NCHUNK = E // CHUNK
CW_LO = NCHUNK // NWORK
N_HI = NCHUNK - CW_LO * NWORK
CW_HI = CW_LO + 1
SLAB = N // 16

_mesh = plsc.VectorSubcoreMesh(core_axis_name="c", subcore_axis_name="s")
_sc_params = pltpu.CompilerParams(use_tc_tiling_on_sc=False)


def _worker_chunks(c, s):
  wid = s * 2 + c
  base = CW_LO * wid + jnp.minimum(wid, N_HI)
  has_extra = wid < N_HI
  return wid, base, has_extra


def _make_deg():
  @functools.partial(
      pl.kernel,
      mesh=_mesh,
      out_type=jax.ShapeDtypeStruct((2, N, 1), jnp.float32),
      compiler_params=_sc_params,
      scratch_types=[
          pltpu.VMEM((CW_HI * CHUNK,), jnp.int32),
          pltpu.VMEM((CHUNK, 1), jnp.float32),
          pltpu.VMEM_SHARED((N, 1), jnp.float32),
          pltpu.SemaphoreType.DMA,
      ],
  )
  def deg_kernel(ei_hbm, ones_hbm, z_hbm, out_hbm, didx, ones_v, acc, ssem):
    c = lax.axis_index("c")
    s = lax.axis_index("s")
    wid, base, has_extra = _worker_chunks(c, s)
    pltpu.sync_copy(z_hbm.at[pl.ds(s * SLAB, SLAB)], acc.at[pl.ds(s * SLAB, SLAB)])
    pltpu.sync_copy(ei_hbm.at[1, pl.ds(base * CHUNK, CW_LO * CHUNK)],
                    didx.at[pl.ds(0, CW_LO * CHUNK)])

    @pl.when(has_extra)
    def _():
      pltpu.sync_copy(
          ei_hbm.at[1, pl.ds((base + CW_LO) * CHUNK, CHUNK)],
          didx.at[pl.ds(CW_LO * CHUNK, CHUNK)])

    pltpu.sync_copy(ones_hbm, ones_v)
    plsc.subcore_barrier()

    def scat_desc(j):
      return pltpu.make_async_copy(
          ones_v, acc.at[didx.at[pl.ds(j * CHUNK, CHUNK)]], ssem)

    for j in range(6):
      scat_desc(j).start(add=True)

    @pl.loop(0, CW_LO - 6)
    def _(j):
      scat_desc(0).wait()
      scat_desc(j + 6).start(add=True)

    @pl.when(has_extra)
    def _():
      scat_desc(0).wait()
      scat_desc(CW_LO).start(add=True)

    @pl.loop(0, 6)
    def _(j):
      scat_desc(0).wait()

    plsc.subcore_barrier()
    pltpu.sync_copy(acc.at[pl.ds(s * SLAB, SLAB)],
                    out_hbm.at[c, pl.ds(s * SLAB, SLAB)])

  return deg_kernel


def _make_agg(d):
  @functools.partial(
      pl.kernel,
      mesh=_mesh,
      out_type=jax.ShapeDtypeStruct((2, N, d), jnp.float32),
      compiler_params=_sc_params,
      scratch_types=[
          pltpu.VMEM((CW_HI * CHUNK,), jnp.int32),
          pltpu.VMEM((CW_HI * CHUNK,), jnp.int32),
          [pltpu.VMEM((CHUNK, d), jnp.float32)] * 4,
          pltpu.VMEM_SHARED((N, d), jnp.float32),
          [pltpu.SemaphoreType.DMA] * 4,
          [pltpu.SemaphoreType.DMA] * 4,
      ],
  )
  def agg_kernel(ei_hbm, g_hbm, z_hbm, out_hbm,
                 sidx, didx, bufs, acc, gsems, ssems):
    c = lax.axis_index("c")
    s = lax.axis_index("s")
    wid, base, has_extra = _worker_chunks(c, s)
    pltpu.sync_copy(z_hbm.at[pl.ds(s * SLAB, SLAB)], acc.at[pl.ds(s * SLAB, SLAB)])
    pltpu.sync_copy(ei_hbm.at[0, pl.ds(base * CHUNK, CW_LO * CHUNK)],
                    sidx.at[pl.ds(0, CW_LO * CHUNK)])
    pltpu.sync_copy(ei_hbm.at[1, pl.ds(base * CHUNK, CW_LO * CHUNK)],
                    didx.at[pl.ds(0, CW_LO * CHUNK)])

    @pl.when(has_extra)
    def _():
      pltpu.sync_copy(ei_hbm.at[0, pl.ds((base + CW_LO) * CHUNK, CHUNK)],
                      sidx.at[pl.ds(CW_LO * CHUNK, CHUNK)])
      pltpu.sync_copy(ei_hbm.at[1, pl.ds((base + CW_LO) * CHUNK, CHUNK)],
                      didx.at[pl.ds(CW_LO * CHUNK, CHUNK)])

    plsc.subcore_barrier()

    def gd(j, i):
      return pltpu.make_async_copy(
          g_hbm.at[sidx.at[pl.ds(j * CHUNK, CHUNK)]], bufs[i], gsems[i])

    def scat(j, i):
      pltpu.sync_copy(bufs[i], acc.at[didx.at[pl.ds(j * CHUNK, CHUNK)]], add=True)

    gd(0, 0).start()
    gd(1, 1).start()

    @pl.loop(0, (CW_LO - 2) // 2)
    def _(jj):
      j0 = jj * 2
      gd(j0, 0).wait()
      scat(j0, 0)
      gd(j0 + 2, 0).start()
      gd(j0 + 1, 1).wait()
      scat(j0 + 1, 1)
      gd(j0 + 3, 1).start()

    gd(CW_LO - 2, 0).wait()
    scat(CW_LO - 2, 0)

    @pl.when(has_extra)
    def _():
      gd(CW_LO, 0).start()

    gd(CW_LO - 1, 1).wait()
    scat(CW_LO - 1, 1)

    @pl.when(has_extra)
    def _():
      gd(CW_LO, 0).wait()
      scat(CW_LO, 0)

    plsc.subcore_barrier()
    pltpu.sync_copy(acc.at[pl.ds(s * SLAB, SLAB)],
                    out_hbm.at[c, pl.ds(s * SLAB, SLAB)])

  return agg_kernel


_deg = _make_deg()
_agg16 = _make_agg(D_HID)
_agg16b = _make_agg(D_HID)


def _mm1_body(x_ref, w_ref, o_ref):
  o_ref[...] = jnp.dot(x_ref[...], w_ref[...],
                       preferred_element_type=jnp.float32)


def _tc_mm1(x, w1):
  return pl.pallas_call(
      _mm1_body,
      grid=(10,),
      in_specs=[
          pl.BlockSpec((N // 10, D_FEAT), lambda i: (i, 0)),
          pl.BlockSpec((D_FEAT, D_HID), lambda i: (0, 0)),
      ],
      out_specs=pl.BlockSpec((N // 10, D_HID), lambda i: (i, 0)),
      out_shape=jax.ShapeDtypeStruct((N, D_HID), jnp.float32),
  )(x, w1)


def _scale1_body(dp_ref, h_ref, dis_ref, g1_ref):
  deg = dp_ref[0] + dp_ref[1] + 1.0
  dis = lax.rsqrt(deg)
  dis_ref[...] = dis
  g1_ref[...] = dis * h_ref[...]


def _tc_scale1(dp, h):
  return pl.pallas_call(
      _scale1_body,
      out_shape=(
          jax.ShapeDtypeStruct((N, 1), jnp.float32),
          jax.ShapeDtypeStruct((N, D_HID), jnp.float32),
      ),
  )(dp, h)


def _mid_body(a_ref, g1_ref, dis_ref, b1_ref, u_ref):
  s = dis_ref[...] * (a_ref[0] + a_ref[1] + g1_ref[...]) + b1_ref[...]
  u_ref[...] = dis_ref[...] * jnp.maximum(s, 0.0)


def _tc_mid(a, g1, dis, b1):
  return pl.pallas_call(
      _mid_body,
      out_shape=jax.ShapeDtypeStruct((N, D_HID), jnp.float32),
  )(a, g1, dis, b1)


def _final_body(q_ref, u_ref, dis_ref, w2_ref, b2_ref, o_ref):
  t = jnp.dot(q_ref[0] + q_ref[1] + u_ref[...], w2_ref[...],
              preferred_element_type=jnp.float32)
  l = dis_ref[...] * t + b2_ref[...]
  m = jnp.max(l, axis=1, keepdims=True)
  e = jnp.exp(l - m)
  lse = jnp.log(jnp.sum(e, axis=1, keepdims=True)) + m
  o_ref[...] = l - lse


def _tc_final(q, u, dis, w2, b2):
  return pl.pallas_call(
      _final_body,
      out_shape=jax.ShapeDtypeStruct((N, N_CLASSES), jnp.float32),
  )(q, u, dis, w2, b2)


def kernel(x, edge_index, W1, b1, W2, b2):
  ei = edge_index.astype(jnp.int32)

  z1 = jnp.zeros((N, 1), jnp.float32)
  z16 = jnp.zeros((N, D_HID), jnp.float32)
  ones = jnp.ones((CHUNK, 1), jnp.float32)

  degp = _deg(ei, ones, z1)
  h = _tc_mm1(x, W1)
  dis, g1 = _tc_scale1(degp, h)
  a = _agg16(ei, g1, z16)
  u = _tc_mid(a, g1, dis, b1.reshape(1, D_HID))
  q = _agg16b(ei, u, z16)
  return _tc_final(q, u, dis, W2, b2.reshape(1, N_CLASSES))
